# baseline (speedup 1.0000x reference)
import functools

import jax
import jax.numpy as jnp
from jax import lax
from jax.experimental import pallas as pl
from jax.experimental.pallas import tpu as pltpu
from jax.experimental.pallas import tpu_sc as plsc

SEQ = 128
D = 64


@jax.jit
def _sc_embed(x, emb_r, pos_t):
    b_total = x.shape[0]
    info = plsc.get_sparse_core_info()
    nc, ns = info.num_cores, info.num_subcores
    nw = nc * ns
    b_per_w = b_total // nw

    mesh = plsc.VectorSubcoreMesh(core_axis_name="c", subcore_axis_name="s")

    @functools.partial(
        pl.kernel,
        mesh=mesh,
        compiler_params=pltpu.CompilerParams(
            use_tc_tiling_on_sc=True, needs_layout_passes=False),
        out_type=jax.ShapeDtypeStruct((b_total, D, SEQ), jnp.float32),
        scratch_types=[
            pltpu.VMEM((b_per_w, SEQ), jnp.int32),
            pltpu.VMEM((b_per_w, SEQ), jnp.int32),
            pltpu.VMEM((b_per_w, SEQ), jnp.int32),
            pltpu.VMEM((D, SEQ), jnp.float32),
            pltpu.VMEM((SEQ, SEQ), jnp.float32),
            pltpu.VMEM((SEQ, SEQ), jnp.float32),
            pltpu.VMEM((D, SEQ), jnp.float32),
            pltpu.VMEM((D, SEQ), jnp.float32),
            pltpu.SemaphoreType.DMA,
            pltpu.SemaphoreType.DMA,
        ],
    )
    def k(x_hbm, emb_hbm, pos_hbm, out_hbm, idx_v, pair_v, sel_v, pos_v,
          buf0, buf1, ob0, ob1, sem0, sem1):
        wid = lax.axis_index("s") * nc + lax.axis_index("c")
        base = wid * b_per_w

        pltpu.sync_copy(x_hbm.at[pl.ds(base, b_per_w)], idx_v)
        pltpu.sync_copy(pos_hbm, pos_v)

        def split_body(r, acc):
            for u in range(8):
                sl = pl.ds(u * 16, 16)
                w = idx_v[r, sl]
                pair_v[r, sl] = lax.shift_right_logical(w, 1)
                sel_v[r, sl] = (w & 1) << 6
            return acc
        lax.fori_loop(0, b_per_w, split_body, 0, unroll=2)

        def start_gather(c, buf, sem):
            pltpu.make_async_copy(emb_hbm.at[pair_v.at[c]], buf, sem).start()

        start_gather(0, buf0, sem0)
        start_gather(1, buf1, sem1)

        iota = lax.iota(jnp.int32, 16)

        def process(c, buf, ob, sem):
            pltpu.make_async_copy(emb_hbm.at[pair_v.at[c]], buf, sem).wait()

            rows = tuple(iota + (16 * kk) for kk in range(8))
            cols = tuple(sel_v[c, pl.ds(16 * kk, 16)] for kk in range(8))

            def v_body(v, carry):
                rws, cls = carry
                for kk in range(8):
                    sl = pl.ds(16 * kk, 16)
                    g = plsc.load_gather(buf, [rws[kk], cls[kk] + v])
                    ob[v, sl] = g + pos_v[v, sl]
                return carry
            lax.fori_loop(0, D, v_body, (rows, cols), unroll=2)

            pltpu.sync_copy(ob, out_hbm.at[base + c])

        def body(i, carry):
            c = i * 2
            process(c, buf0, ob0, sem0)
            start_gather(lax.rem(c + 2, b_per_w), buf0, sem0)
            process(c + 1, buf1, ob1, sem1)
            start_gather(lax.rem(c + 3, b_per_w), buf1, sem1)
            return carry

        lax.fori_loop(0, b_per_w // 2, body, 0)

        pltpu.make_async_copy(emb_hbm.at[pair_v.at[0]], buf0, sem0).wait()
        pltpu.make_async_copy(emb_hbm.at[pair_v.at[1]], buf1, sem1).wait()

    return k(x, emb_r, pos_t)


def kernel(x, embedding, pos_embedding):
    b, s = x.shape
    emb_r = embedding.reshape(embedding.shape[0] // 2, 2 * D)
    out_t = _sc_embed(x.astype(jnp.int32), emb_r, pos_embedding.T)
    return out_t.transpose(0, 2, 1)


# speedup vs baseline: 1.0023x; 1.0023x over previous
import functools

import jax
import jax.numpy as jnp
from jax import lax
from jax.experimental import pallas as pl
from jax.experimental.pallas import tpu as pltpu
from jax.experimental.pallas import tpu_sc as plsc

SEQ = 128
D = 64


@jax.jit
def _sc_embed(x, emb, pos_t):
    b_total = x.shape[0]
    info = plsc.get_sparse_core_info()
    nc, ns = info.num_cores, info.num_subcores
    nw = nc * ns
    b_per_w = b_total // nw

    mesh = plsc.VectorSubcoreMesh(core_axis_name="c", subcore_axis_name="s")

    @functools.partial(
        pl.kernel,
        mesh=mesh,
        compiler_params=pltpu.CompilerParams(
            use_tc_tiling_on_sc=False, needs_layout_passes=False),
        out_type=jax.ShapeDtypeStruct((b_total, D, SEQ), jnp.float32),
        scratch_types=[
            pltpu.VMEM((b_per_w, SEQ), jnp.int32),
            pltpu.VMEM((D, SEQ), jnp.float32),
            pltpu.VMEM((SEQ, D), jnp.float32),
            pltpu.VMEM((SEQ, D), jnp.float32),
            pltpu.VMEM((D, SEQ), jnp.float32),
            pltpu.VMEM((D, SEQ), jnp.float32),
            pltpu.SemaphoreType.DMA,
            pltpu.SemaphoreType.DMA,
        ],
    )
    def k(x_hbm, emb_hbm, pos_hbm, out_hbm, idx_v, pos_v,
          buf0, buf1, ob0, ob1, sem0, sem1):
        wid = lax.axis_index("s") * nc + lax.axis_index("c")
        base = wid * b_per_w

        pltpu.sync_copy(x_hbm.at[pl.ds(base, b_per_w)], idx_v)
        pltpu.sync_copy(pos_hbm, pos_v)

        def start_gather(c, buf, sem):
            pltpu.make_async_copy(emb_hbm.at[idx_v.at[c]], buf, sem).start()

        start_gather(0, buf0, sem0)
        start_gather(1, buf1, sem1)

        iota = lax.iota(jnp.int32, 16)

        def process(c, buf, ob, sem):
            pltpu.make_async_copy(emb_hbm.at[idx_v.at[c]], buf, sem).wait()

            def v_body(v, acc):
                cols = jnp.full((16,), 0, jnp.int32) + v
                for kk in range(8):
                    sl = pl.ds(16 * kk, 16)
                    g = plsc.load_gather(buf, [iota + (16 * kk), cols])
                    ob[v, sl] = g + pos_v[v, sl]
                return acc
            lax.fori_loop(0, D, v_body, 0, unroll=4)

            pltpu.sync_copy(ob, out_hbm.at[base + c])

        def body(i, carry):
            c = i * 2
            process(c, buf0, ob0, sem0)
            start_gather(lax.rem(c + 2, b_per_w), buf0, sem0)
            process(c + 1, buf1, ob1, sem1)
            start_gather(lax.rem(c + 3, b_per_w), buf1, sem1)
            return carry

        lax.fori_loop(0, b_per_w // 2, body, 0)

        pltpu.make_async_copy(emb_hbm.at[idx_v.at[0]], buf0, sem0).wait()
        pltpu.make_async_copy(emb_hbm.at[idx_v.at[1]], buf1, sem1).wait()

    return k(x, emb, pos_t)


def kernel(x, embedding, pos_embedding):
    out_t = _sc_embed(x.astype(jnp.int32), embedding, pos_embedding.T)
    return out_t.transpose(0, 2, 1)


# speedup vs baseline: 1.4634x; 1.4601x over previous
import functools

import jax
import jax.numpy as jnp
from jax import lax
from jax.experimental import pallas as pl
from jax.experimental.pallas import tpu as pltpu
from jax.experimental.pallas import tpu_sc as plsc

SEQ = 128
D = 64


@jax.jit
def _sc_embed(x, emb, pos_t):
    b_total = x.shape[0]
    info = plsc.get_sparse_core_info()
    nc, ns = info.num_cores, info.num_subcores
    nw = nc * ns
    b_per_w = b_total // nw

    mesh = plsc.VectorSubcoreMesh(core_axis_name="c", subcore_axis_name="s")

    @functools.partial(
        pl.kernel,
        mesh=mesh,
        compiler_params=pltpu.CompilerParams(
            use_tc_tiling_on_sc=False, needs_layout_passes=False),
        out_type=jax.ShapeDtypeStruct((b_total, D, SEQ), jnp.float32),
        scratch_types=[
            pltpu.VMEM((b_per_w, SEQ), jnp.int32),
            pltpu.VMEM((D, SEQ), jnp.float32),
            pltpu.VMEM((SEQ, D), jnp.float32),
            pltpu.VMEM((SEQ, D), jnp.float32),
            pltpu.VMEM((D, SEQ), jnp.float32),
            pltpu.VMEM((D, SEQ), jnp.float32),
            pltpu.SemaphoreType.DMA,
            pltpu.SemaphoreType.DMA,
        ],
    )
    def k(x_hbm, emb_hbm, pos_hbm, out_hbm, idx_v, pos_v,
          buf0, buf1, ob0, ob1, sem0, sem1):
        wid = lax.axis_index("s") * nc + lax.axis_index("c")
        base = wid * b_per_w

        pltpu.sync_copy(x_hbm.at[pl.ds(base, b_per_w)], idx_v)
        pltpu.sync_copy(pos_hbm, pos_v)

        def start_gather(c, buf, sem):
            pltpu.make_async_copy(emb_hbm.at[idx_v.at[c]], buf, sem).start()

        start_gather(0, buf0, sem0)
        start_gather(1, buf1, sem1)

        iota = lax.iota(jnp.int32, 16)

        def process(c, buf, ob, sem):
            pltpu.make_async_copy(emb_hbm.at[idx_v.at[c]], buf, sem).wait()

            def t_body(i, acc):
                j = i & 15
                blk = i >> 4
                kk = blk & 7
                vb = blk >> 3
                rows = iota + (16 * kk)
                vr = ((iota + j) & 15) + (16 * vb)
                g = plsc.load_gather(buf, [rows, vr])
                p = plsc.load_gather(pos_v, [vr, rows])
                plsc.store_scatter(ob, [vr, rows], g + p)
                return acc
            lax.fori_loop(0, 512, t_body, 0, unroll=2)

            pltpu.sync_copy(ob, out_hbm.at[base + c])

        def body(i, carry):
            c = i * 2
            process(c, buf0, ob0, sem0)
            start_gather(lax.rem(c + 2, b_per_w), buf0, sem0)
            process(c + 1, buf1, ob1, sem1)
            start_gather(lax.rem(c + 3, b_per_w), buf1, sem1)
            return carry

        lax.fori_loop(0, b_per_w // 2, body, 0)

        pltpu.make_async_copy(emb_hbm.at[idx_v.at[0]], buf0, sem0).wait()
        pltpu.make_async_copy(emb_hbm.at[idx_v.at[1]], buf1, sem1).wait()

    return k(x, emb, pos_t)


def kernel(x, embedding, pos_embedding):
    out_t = _sc_embed(x.astype(jnp.int32), embedding, pos_embedding.T)
    return out_t.transpose(0, 2, 1)


# speedup vs baseline: 1.5315x; 1.0465x over previous
import functools

import jax
import jax.numpy as jnp
from jax import lax
from jax.experimental import pallas as pl
from jax.experimental.pallas import tpu as pltpu
from jax.experimental.pallas import tpu_sc as plsc

SEQ = 128
D = 64


@jax.jit
def _sc_embed(x, emb, pos_t):
    b_total = x.shape[0]
    info = plsc.get_sparse_core_info()
    nc, ns = info.num_cores, info.num_subcores
    nw = nc * ns
    b_per_w = b_total // nw

    mesh = plsc.VectorSubcoreMesh(core_axis_name="c", subcore_axis_name="s")

    @functools.partial(
        pl.kernel,
        mesh=mesh,
        compiler_params=pltpu.CompilerParams(
            use_tc_tiling_on_sc=False, needs_layout_passes=False),
        out_type=jax.ShapeDtypeStruct((b_total, D, SEQ), jnp.float32),
        scratch_types=[
            pltpu.VMEM((b_per_w, SEQ), jnp.int32),
            pltpu.VMEM((D, SEQ), jnp.float32),
            pltpu.VMEM((SEQ, D), jnp.float32),
            pltpu.VMEM((SEQ, D), jnp.float32),
            pltpu.VMEM((D, SEQ), jnp.float32),
            pltpu.VMEM((D, SEQ), jnp.float32),
            pltpu.SemaphoreType.DMA,
            pltpu.SemaphoreType.DMA,
        ],
    )
    def k(x_hbm, emb_hbm, pos_hbm, out_hbm, idx_v, pos_v,
          buf0, buf1, ob0, ob1, sem0, sem1):
        wid = lax.axis_index("s") * nc + lax.axis_index("c")
        base = wid * b_per_w

        pltpu.sync_copy(x_hbm.at[pl.ds(base, b_per_w)], idx_v)
        pltpu.sync_copy(pos_hbm, pos_v)

        def start_gather(c, buf, sem):
            pltpu.make_async_copy(emb_hbm.at[idx_v.at[c]], buf, sem).start()

        start_gather(0, buf0, sem0)
        start_gather(1, buf1, sem1)

        iota = lax.iota(jnp.int32, 16)

        def process(c, buf, ob, sem):
            pltpu.make_async_copy(emb_hbm.at[idx_v.at[c]], buf, sem).wait()

            def t_body(blk, acc):
                kk16 = (blk & 7) * 16
                vb16 = (blk >> 3) * 16
                rows = iota + kk16
                for j in range(16):
                    diag = (iota + j) & 15
                    vr = diag + vb16
                    g = plsc.load_gather(buf, [rows, vr])
                    p = plsc.load_gather(pos_v, [vr, rows])
                    plsc.store_scatter(ob, [vr, rows], g + p)
                return acc
            lax.fori_loop(0, 32, t_body, 0, unroll=2)

            pltpu.sync_copy(ob, out_hbm.at[base + c])

        def body(i, carry):
            c = i * 2
            process(c, buf0, ob0, sem0)
            start_gather(lax.rem(c + 2, b_per_w), buf0, sem0)
            process(c + 1, buf1, ob1, sem1)
            start_gather(lax.rem(c + 3, b_per_w), buf1, sem1)
            return carry

        lax.fori_loop(0, b_per_w // 2, body, 0)

        pltpu.make_async_copy(emb_hbm.at[idx_v.at[0]], buf0, sem0).wait()
        pltpu.make_async_copy(emb_hbm.at[idx_v.at[1]], buf1, sem1).wait()

    return k(x, emb, pos_t)


def kernel(x, embedding, pos_embedding):
    out_t = _sc_embed(x.astype(jnp.int32), embedding, pos_embedding.T)
    return out_t.transpose(0, 2, 1)
